# Initial kernel scaffold; baseline (speedup 1.0000x reference)
import jax
import jax.numpy as jnp
from jax.experimental import pallas as pl


def kernel(x, edge_index, batch, W1, att_src1, att_dst1, b1, W2, att_src2, att_dst2, b2, lin1_W, lin1_b, lin2_W, lin2_b):
    raise NotImplementedError("write your pallas kernel here")


# speedup vs baseline: 39.2121x; 39.2121x over previous
import functools

import jax
import jax.numpy as jnp
from jax import lax
from jax.experimental import pallas as pl
from jax.experimental.pallas import tpu as pltpu
from jax.experimental.pallas import tpu_sc as plsc

NN = 10000
EE = 320000
NG = 32
F32 = jnp.float32

NC, NS, L = 2, 16, 16
EPT = EE // NS
CHUNK = 80
NCHUNK = EPT // CHUNK
TW1 = 40
TW2 = 32
AW = 48
NP = 10240
RPT = NP // NS
ZR = 128


def _tca_body(x_ref, w1_ref, as_ref, ad_ref, tab_ref, adst_ref):
  xw = jnp.dot(x_ref[...], w1_ref[...], preferred_element_type=F32)
  asrc = jnp.dot(xw, as_ref[...], preferred_element_type=F32, precision=lax.Precision.HIGHEST)
  tab_ref[0] = jnp.concatenate([xw[:, 0:32], asrc], axis=1)
  tab_ref[1] = jnp.concatenate([xw[:, 32:64], asrc], axis=1)
  adst_ref[...] = jnp.dot(xw, ad_ref[...], preferred_element_type=F32, precision=lax.Precision.HIGHEST)


def _tcb_body(q_ref, b1r, b8r, w2_ref, was2_ref, wad2_ref,
              tab_ref, asrc2_ref, adst2_ref):
  num = jnp.concatenate([q_ref[0, :, 0:32], q_ref[1, :, 0:32]], axis=1)
  den = q_ref[0, :, 32:40]
  denb = jnp.dot(den, b8r[...], preferred_element_type=F32, precision=lax.Precision.HIGHEST)
  v = num / (denb + 1e-16) + b1r[...]
  h1o = jnp.where(v > 0, v, jnp.exp(v) - 1.0)
  h2 = jnp.dot(h1o, w2_ref[...], preferred_element_type=F32)
  tab_ref[0] = h2[:, 0:32]
  tab_ref[1] = h2[:, 32:64]
  asrc2_ref[...] = jnp.sum(h2 * was2_ref[...], axis=1, keepdims=True)
  adst2_ref[...] = jnp.sum(h2 * wad2_ref[...], axis=1, keepdims=True)


def _tcc_body(q_ref, batch_ref, b2r, l1w, l1b, l2w, l2b, out_ref,
              pool_ref, cnt_ref):
  i = pl.program_id(0)

  @pl.when(i == 0)
  def _():
    pool_ref[...] = jnp.zeros((NG, 64), F32)
    cnt_ref[...] = jnp.zeros((NG, 1), F32)

  num = jnp.concatenate([q_ref[0, :, 0:32], q_ref[1, :, 0:32]], axis=1)
  den = q_ref[0, :, 32:33]
  h2o = num / (den + 1e-16) + b2r[...]
  b = batch_ref[...]
  gid = lax.broadcasted_iota(jnp.int32, (1, NG), 1)
  ot = (b == gid).astype(F32)
  dn = (((0,), (0,)), ((), ()))
  pool_ref[...] += lax.dot_general(ot, h2o, dn, preferred_element_type=F32, precision=lax.Precision.HIGHEST)
  cnt_ref[...] += lax.dot_general(ot, jnp.ones((h2o.shape[0], 1), F32), dn,
                                  preferred_element_type=F32, precision=lax.Precision.HIGHEST)

  @pl.when(i == pl.num_programs(0) - 1)
  def _():
    mean = pool_ref[...] / jnp.maximum(cnt_ref[...], 1.0)
    o1 = jnp.dot(mean, l1w[...], preferred_element_type=F32) + l1b[...]
    o1 = jnp.where(o1 > 0, o1, jnp.exp(o1) - 1.0)
    out_ref[...] = jnp.dot(o1, l2w[...], preferred_element_type=F32) + l2b[...]


def _dg(v, idx):
  return v.at[idx].get(mode="promise_in_bounds")


def _zero_acc(zbuf, acc, base):
  zv = jnp.zeros((L,), F32)
  def zrow(r, _):
    for k in range(AW // L):
      zbuf[r, pl.ds(k * L, L)] = zv
    return 0
  lax.fori_loop(0, ZR, zrow, 0)
  for k in range(RPT // ZR):
    pltpu.sync_copy(zbuf, acc.at[pl.ds(base + k * ZR, ZR), :])


def _dump_acc(zbuf, acc, out_hbm, cid, base):
  for k in range(RPT // ZR):
    pltpu.sync_copy(acc.at[pl.ds(base + k * ZR, ZR), :], zbuf)
    pltpu.sync_copy(zbuf, out_hbm.at[cid, pl.ds(base + k * ZR, ZR), :])


_sc_mesh = plsc.VectorSubcoreMesh(core_axis_name="c", subcore_axis_name="s")
_sc_params = pltpu.CompilerParams(needs_layout_passes=False,
                                  use_tc_tiling_on_sc=False)


@functools.partial(
    pl.kernel,
    out_type=pltpu.HBM((NC, NP, AW), F32),
    mesh=_sc_mesh,
    scratch_types=[
        pltpu.VMEM((CHUNK,), jnp.int32),
        pltpu.VMEM((CHUNK,), jnp.int32),
        pltpu.VMEM((CHUNK, TW1), F32),
        pltpu.VMEM((CHUNK, AW), F32),
        pltpu.VMEM((NN * 8,), F32),
        pltpu.VMEM((ZR, AW), F32),
        pltpu.VMEM_SHARED((NP, AW), F32),
        pltpu.SemaphoreType.DMA,
    ],
    compiler_params=_sc_params,
)
def _sc_layer1(tab_hbm, adst1_hbm, src_hbm, dst_hbm, out,
               srcv, dstv, rows, outv, adres, zbuf, acc, sem):
  cid = lax.axis_index("c")
  sid = lax.axis_index("s")
  lane = lax.iota(jnp.int32, L)
  row2 = lane >> 3
  col8 = lane & 7

  _zero_acc(zbuf, acc, sid * RPT)
  pltpu.sync_copy(adst1_hbm, adres)
  plsc.subcore_barrier()

  def chunk(ci, _):
    ebase = sid * EPT + ci * CHUNK
    pltpu.sync_copy(src_hbm.at[pl.ds(ebase, CHUNK)], srcv)
    pltpu.sync_copy(dst_hbm.at[pl.ds(ebase, CHUNK)], dstv)
    off = cid * NN
    for j in range(CHUNK // L):
      srcv[pl.ds(j * L, L)] = srcv[pl.ds(j * L, L)] + off
    pltpu.async_copy(tab_hbm.at[srcv], rows, sem).wait()
    for g in range(CHUNK // L):
      d16 = dstv[pl.ds(g * L, L)]
      for p in range(L // 2):
        e0 = g * L + 2 * p
        rid = _dg(d16, row2 + 2 * p)
        adv = plsc.load_gather(adres, [rid * 8 + col8])
        va = rows[e0, pl.ds(24, L)]
        vb = rows[e0 + 1, pl.ds(24, L)]
        asv = jnp.where(lane < 8, _dg(va, col8 + 8), vb)
        alpha = asv + adv
        alpha = jnp.where(alpha > 0, alpha, 0.2 * alpha)
        ex = jnp.exp(alpha)
        for k in range(2):
          bA = _dg(ex, row2 + (2 * k + 4 * cid))
          outv[e0, pl.ds(k * L, L)] = rows[e0, pl.ds(k * L, L)] * bA
          bB = _dg(ex, row2 + (2 * k + 4 * cid + 8))
          outv[e0 + 1, pl.ds(k * L, L)] = rows[e0 + 1, pl.ds(k * L, L)] * bB
        outv[e0, pl.ds(32, L)] = jnp.where(lane < 8, ex, 0.0)
        exb = _dg(ex, col8 + 8)
        outv[e0 + 1, pl.ds(32, L)] = jnp.where(lane < 8, exb, 0.0)
    pltpu.sync_copy(outv, acc.at[dstv], add=True)
    return 0

  lax.fori_loop(0, NCHUNK, chunk, 0)
  plsc.subcore_barrier()
  _dump_acc(zbuf, acc, out, cid, sid * RPT)


@functools.partial(
    pl.kernel,
    out_type=pltpu.HBM((NC, NP, AW), F32),
    mesh=_sc_mesh,
    scratch_types=[
        pltpu.VMEM((CHUNK,), jnp.int32),
        pltpu.VMEM((CHUNK,), jnp.int32),
        pltpu.VMEM((CHUNK, TW2), F32),
        pltpu.VMEM((CHUNK, AW), F32),
        pltpu.VMEM((NN,), F32),
        pltpu.VMEM((NN,), F32),
        pltpu.VMEM((ZR, AW), F32),
        pltpu.VMEM_SHARED((NP, AW), F32),
        pltpu.SemaphoreType.DMA,
    ],
    compiler_params=_sc_params,
)
def _sc_layer2(tab_hbm, asrc2_hbm, adst2_hbm, src_hbm, dst_hbm, out,
               srcv, dstv, rows, outv, asres, adres, zbuf, acc, sem):
  cid = lax.axis_index("c")
  sid = lax.axis_index("s")
  lane = lax.iota(jnp.int32, L)
  zero16 = lane * 0

  _zero_acc(zbuf, acc, sid * RPT)
  pltpu.sync_copy(asrc2_hbm, asres)
  pltpu.sync_copy(adst2_hbm, adres)
  plsc.subcore_barrier()

  def chunk(ci, _):
    ebase = sid * EPT + ci * CHUNK
    pltpu.sync_copy(src_hbm.at[pl.ds(ebase, CHUNK)], srcv)
    pltpu.sync_copy(dst_hbm.at[pl.ds(ebase, CHUNK)], dstv)
    off = cid * NN
    for j in range(CHUNK // L):
      srcv[pl.ds(j * L, L)] = srcv[pl.ds(j * L, L)] + off
    pltpu.async_copy(tab_hbm.at[srcv], rows, sem).wait()
    for g in range(CHUNK // L):
      s16 = srcv[pl.ds(g * L, L)] - off
      d16 = dstv[pl.ds(g * L, L)]
      adv = plsc.load_gather(adres, [d16])
      asv = plsc.load_gather(asres, [s16])
      alpha = asv + adv
      alpha = jnp.where(alpha > 0, alpha, 0.2 * alpha)
      ex = jnp.exp(alpha)
      for t in range(L):
        e = g * L + t
        b = _dg(ex, zero16 + t)
        for k in range(2):
          outv[e, pl.ds(k * L, L)] = rows[e, pl.ds(k * L, L)] * b
        outv[e, pl.ds(32, L)] = jnp.where(lane < 1, b, 0.0)
    pltpu.sync_copy(outv, acc.at[dstv], add=True)
    return 0

  lax.fori_loop(0, NCHUNK, chunk, 0)
  plsc.subcore_barrier()
  _dump_acc(zbuf, acc, out, cid, sid * RPT)


def kernel(x, edge_index, batch, W1, att_src1, att_dst1, b1,
           W2, att_src2, att_dst2, b2, lin1_W, lin1_b, lin2_W, lin2_b):
  src = edge_index[0]
  dst = edge_index[1]
  eye8 = jnp.eye(8, dtype=F32)
  as1 = (att_src1[0][:, :, None] * eye8[:, None, :]).reshape(64, 8)
  ad1 = (att_dst1[0][:, :, None] * eye8[:, None, :]).reshape(64, 8)
  was2 = att_src2[0, 0].reshape(1, 64)
  wad2 = att_dst2[0, 0].reshape(1, 64)
  b8 = jnp.repeat(eye8, 8, axis=1)

  RB = 2000
  tab1, adst1 = pl.pallas_call(
      _tca_body,
      grid=(NN // RB,),
      in_specs=[pl.BlockSpec((RB, 128), lambda i: (i, 0)),
                pl.BlockSpec((128, 64), lambda i: (0, 0)),
                pl.BlockSpec((64, 8), lambda i: (0, 0)),
                pl.BlockSpec((64, 8), lambda i: (0, 0))],
      out_specs=[pl.BlockSpec((2, RB, TW1), lambda i: (0, i, 0)),
                 pl.BlockSpec((RB, 8), lambda i: (i, 0))],
      out_shape=[jax.ShapeDtypeStruct((2, NN, TW1), F32),
                 jax.ShapeDtypeStruct((NN, 8), F32)],
  )(x, W1, as1, ad1)

  p1 = _sc_layer1(tab1.reshape(2 * NN, TW1), adst1.reshape(NN * 8), src, dst)

  tab2, asrc2, adst2 = pl.pallas_call(
      _tcb_body,
      grid=(NN // RB,),
      in_specs=[pl.BlockSpec((2, RB, AW), lambda i: (0, i, 0)),
                pl.BlockSpec((1, 64), lambda i: (0, 0)),
                pl.BlockSpec((8, 64), lambda i: (0, 0)),
                pl.BlockSpec((64, 64), lambda i: (0, 0)),
                pl.BlockSpec((1, 64), lambda i: (0, 0)),
                pl.BlockSpec((1, 64), lambda i: (0, 0))],
      out_specs=[pl.BlockSpec((2, RB, TW2), lambda i: (0, i, 0)),
                 pl.BlockSpec((RB, 1), lambda i: (i, 0)),
                 pl.BlockSpec((RB, 1), lambda i: (i, 0))],
      out_shape=[jax.ShapeDtypeStruct((2, NN, TW2), F32),
                 jax.ShapeDtypeStruct((NN, 1), F32),
                 jax.ShapeDtypeStruct((NN, 1), F32)],
  )(p1, b1.reshape(1, 64), b8, W2, was2, wad2)

  p2 = _sc_layer2(tab2.reshape(2 * NN, TW2), asrc2.reshape(NN),
                  adst2.reshape(NN), src, dst)

  out = pl.pallas_call(
      _tcc_body,
      grid=(NN // RB,),
      in_specs=[pl.BlockSpec((2, RB, AW), lambda i: (0, i, 0)),
                pl.BlockSpec((RB, 1), lambda i: (i, 0)),
                pl.BlockSpec((1, 64), lambda i: (0, 0)),
                pl.BlockSpec((64, 128), lambda i: (0, 0)),
                pl.BlockSpec((1, 128), lambda i: (0, 0)),
                pl.BlockSpec((128, 1), lambda i: (0, 0)),
                pl.BlockSpec((1, 1), lambda i: (0, 0))],
      out_specs=pl.BlockSpec((NG, 1), lambda i: (0, 0)),
      out_shape=jax.ShapeDtypeStruct((NG, 1), F32),
      scratch_shapes=[pltpu.VMEM((NG, 64), F32), pltpu.VMEM((NG, 1), F32)],
  )(p2, batch.reshape(NN, 1), b2.reshape(1, 64),
    lin1_W, lin1_b.reshape(1, 128), lin2_W, lin2_b.reshape(1, 1))
  return out


# speedup vs baseline: 48.3161x; 1.2322x over previous
import functools

import jax
import jax.numpy as jnp
from jax import lax
from jax.experimental import pallas as pl
from jax.experimental.pallas import tpu as pltpu
from jax.experimental.pallas import tpu_sc as plsc

NN = 10000
EE = 320000
NG = 32
F32 = jnp.float32

NC, NS, L = 2, 16, 16
EPT = EE // NS
CHUNK = 80
NCHUNK = EPT // CHUNK
LOOPN = NCHUNK // 2
TW1 = 40
TW2 = 32
AW = 48
NP = 10240
RPT = NP // NS
ZR = 128


def _tca_body(x_ref, w1_ref, as_ref, ad_ref, tab_ref, adst_ref):
  xw = jnp.dot(x_ref[...], w1_ref[...], preferred_element_type=F32)
  asrc = jnp.dot(xw, as_ref[...], preferred_element_type=F32, precision=lax.Precision.HIGHEST)
  tab_ref[0] = jnp.concatenate([xw[:, 0:32], asrc], axis=1)
  tab_ref[1] = jnp.concatenate([xw[:, 32:64], asrc], axis=1)
  adst_ref[...] = jnp.dot(xw, ad_ref[...], preferred_element_type=F32, precision=lax.Precision.HIGHEST)


def _tcb_body(q_ref, b1r, b8r, w2_ref, was2_ref, wad2_ref,
              tab_ref, asrc2_ref, adst2_ref):
  num = jnp.concatenate([q_ref[0, :, 0:32], q_ref[1, :, 0:32]], axis=1)
  den = q_ref[0, :, 32:40]
  denb = jnp.dot(den, b8r[...], preferred_element_type=F32, precision=lax.Precision.HIGHEST)
  v = num / (denb + 1e-16) + b1r[...]
  h1o = jnp.where(v > 0, v, jnp.exp(v) - 1.0)
  h2 = jnp.dot(h1o, w2_ref[...], preferred_element_type=F32)
  tab_ref[0] = h2[:, 0:32]
  tab_ref[1] = h2[:, 32:64]
  asrc2_ref[...] = jnp.sum(h2 * was2_ref[...], axis=1, keepdims=True)
  adst2_ref[...] = jnp.sum(h2 * wad2_ref[...], axis=1, keepdims=True)


def _tcc_body(q_ref, batch_ref, b2r, l1w, l1b, l2w, l2b, out_ref,
              pool_ref, cnt_ref):
  i = pl.program_id(0)

  @pl.when(i == 0)
  def _():
    pool_ref[...] = jnp.zeros((NG, 64), F32)
    cnt_ref[...] = jnp.zeros((NG, 1), F32)

  num = jnp.concatenate([q_ref[0, :, 0:32], q_ref[1, :, 0:32]], axis=1)
  den = q_ref[0, :, 32:33]
  h2o = num / (den + 1e-16) + b2r[...]
  b = batch_ref[...]
  gid = lax.broadcasted_iota(jnp.int32, (1, NG), 1)
  ot = (b == gid).astype(F32)
  dn = (((0,), (0,)), ((), ()))
  pool_ref[...] += lax.dot_general(ot, h2o, dn, preferred_element_type=F32, precision=lax.Precision.HIGHEST)
  cnt_ref[...] += lax.dot_general(ot, jnp.ones((h2o.shape[0], 1), F32), dn,
                                  preferred_element_type=F32, precision=lax.Precision.HIGHEST)

  @pl.when(i == pl.num_programs(0) - 1)
  def _():
    mean = pool_ref[...] / jnp.maximum(cnt_ref[...], 1.0)
    o1 = jnp.dot(mean, l1w[...], preferred_element_type=F32) + l1b[...]
    o1 = jnp.where(o1 > 0, o1, jnp.exp(o1) - 1.0)
    out_ref[...] = jnp.dot(o1, l2w[...], preferred_element_type=F32) + l2b[...]


def _dg(v, idx):
  return v.at[idx].get(mode="promise_in_bounds")


def _zero_acc(zbuf, acc, base):
  zv = jnp.zeros((L,), F32)
  def zrow(r, _):
    for k in range(AW // L):
      zbuf[r, pl.ds(k * L, L)] = zv
    return 0
  lax.fori_loop(0, ZR, zrow, 0)
  for k in range(RPT // ZR):
    pltpu.sync_copy(zbuf, acc.at[pl.ds(base + k * ZR, ZR), :])


def _dump_acc(zbuf, acc, out_hbm, cid, base):
  for k in range(RPT // ZR):
    pltpu.sync_copy(acc.at[pl.ds(base + k * ZR, ZR), :], zbuf)
    pltpu.sync_copy(zbuf, out_hbm.at[cid, pl.ds(base + k * ZR, ZR), :])


_sc_mesh = plsc.VectorSubcoreMesh(core_axis_name="c", subcore_axis_name="s")
_sc_params = pltpu.CompilerParams(needs_layout_passes=False,
                                  use_tc_tiling_on_sc=False)


@functools.partial(
    pl.kernel,
    out_type=pltpu.HBM((NC, NP, AW), F32),
    mesh=_sc_mesh,
    scratch_types=[
        pltpu.VMEM((2, CHUNK), jnp.int32),
        pltpu.VMEM((2, CHUNK), jnp.int32),
        pltpu.VMEM((CHUNK, TW1), F32),
        pltpu.VMEM((CHUNK, TW1), F32),
        pltpu.VMEM((CHUNK, AW), F32),
        pltpu.VMEM((NN * 8,), F32),
        pltpu.VMEM((ZR, AW), F32),
        pltpu.VMEM_SHARED((NP, AW), F32),
        pltpu.SemaphoreType.DMA,
        pltpu.SemaphoreType.DMA,
    ],
    compiler_params=_sc_params,
)
def _sc_layer1(tab_hbm, adst1_hbm, src_hbm, dst_hbm, out,
               srcv, dstv, rows0, rows1, outv, adres, zbuf, acc, sg0, sg1):
  cid = lax.axis_index("c")
  sid = lax.axis_index("s")
  lane = lax.iota(jnp.int32, L)
  row2 = lane >> 3
  col8 = lane & 7
  rows = (rows0, rows1)
  sg = (sg0, sg1)
  tbase = sid * EPT
  off = cid * NN

  _zero_acc(zbuf, acc, sid * RPT)
  pltpu.sync_copy(adst1_hbm, adres)
  plsc.subcore_barrier()

  def prefetch(c, b):
    eb = tbase + c * CHUNK
    pltpu.sync_copy(src_hbm.at[pl.ds(eb, CHUNK)], srcv.at[b])
    pltpu.sync_copy(dst_hbm.at[pl.ds(eb, CHUNK)], dstv.at[b])
    for j in range(CHUNK // L):
      srcv[b, pl.ds(j * L, L)] = srcv[b, pl.ds(j * L, L)] + off
    pltpu.async_copy(tab_hbm.at[srcv.at[b]], rows[b], sg[b])

  def gwait(b):
    pltpu.make_async_copy(tab_hbm.at[pl.ds(0, CHUNK)], rows[b], sg[b]).wait()

  def compute(b):
    rw = rows[b]
    for g in range(CHUNK // L):
      d16 = dstv[b, pl.ds(g * L, L)]
      for p in range(L // 2):
        e0 = g * L + 2 * p
        rid = _dg(d16, row2 + 2 * p)
        adv = plsc.load_gather(adres, [rid * 8 + col8])
        va = rw[e0, pl.ds(24, L)]
        vb = rw[e0 + 1, pl.ds(24, L)]
        asv = jnp.where(lane < 8, _dg(va, col8 + 8), vb)
        alpha = asv + adv
        alpha = jnp.where(alpha > 0, alpha, 0.2 * alpha)
        ex = jnp.exp(alpha)
        for k in range(2):
          bA = _dg(ex, row2 + (2 * k + 4 * cid))
          outv[e0, pl.ds(k * L, L)] = rw[e0, pl.ds(k * L, L)] * bA
          bB = _dg(ex, row2 + (2 * k + 4 * cid + 8))
          outv[e0 + 1, pl.ds(k * L, L)] = rw[e0 + 1, pl.ds(k * L, L)] * bB
        outv[e0, pl.ds(32, L)] = jnp.where(lane < 8, ex, 0.0)
        exb = _dg(ex, col8 + 8)
        outv[e0 + 1, pl.ds(32, L)] = jnp.where(lane < 8, exb, 0.0)
    pltpu.sync_copy(outv, acc.at[dstv.at[b]], add=True)

  prefetch(0, 0)

  def it(i, _):
    prefetch(2 * i + 1, 1)
    gwait(0)
    compute(0)

    @pl.when(i < LOOPN - 1)
    def _():
      prefetch(2 * i + 2, 0)

    gwait(1)
    compute(1)
    return 0

  lax.fori_loop(0, LOOPN, it, 0)
  plsc.subcore_barrier()
  _dump_acc(zbuf, acc, out, cid, sid * RPT)


@functools.partial(
    pl.kernel,
    out_type=pltpu.HBM((NC, NP, AW), F32),
    mesh=_sc_mesh,
    scratch_types=[
        pltpu.VMEM((2, CHUNK), jnp.int32),
        pltpu.VMEM((2, CHUNK), jnp.int32),
        pltpu.VMEM((CHUNK, TW2), F32),
        pltpu.VMEM((CHUNK, TW2), F32),
        pltpu.VMEM((CHUNK, AW), F32),
        pltpu.VMEM((NN,), F32),
        pltpu.VMEM((NN,), F32),
        pltpu.VMEM((ZR, AW), F32),
        pltpu.VMEM_SHARED((NP, AW), F32),
        pltpu.SemaphoreType.DMA,
        pltpu.SemaphoreType.DMA,
    ],
    compiler_params=_sc_params,
)
def _sc_layer2(tab_hbm, asrc2_hbm, adst2_hbm, src_hbm, dst_hbm, out,
               srcv, dstv, rows0, rows1, outv, asres, adres, zbuf, acc,
               sg0, sg1):
  cid = lax.axis_index("c")
  sid = lax.axis_index("s")
  lane = lax.iota(jnp.int32, L)
  zero16 = lane * 0
  rows = (rows0, rows1)
  sg = (sg0, sg1)
  tbase = sid * EPT
  off = cid * NN

  _zero_acc(zbuf, acc, sid * RPT)
  pltpu.sync_copy(asrc2_hbm, asres)
  pltpu.sync_copy(adst2_hbm, adres)
  plsc.subcore_barrier()

  def prefetch(c, b):
    eb = tbase + c * CHUNK
    pltpu.sync_copy(src_hbm.at[pl.ds(eb, CHUNK)], srcv.at[b])
    pltpu.sync_copy(dst_hbm.at[pl.ds(eb, CHUNK)], dstv.at[b])
    for j in range(CHUNK // L):
      srcv[b, pl.ds(j * L, L)] = srcv[b, pl.ds(j * L, L)] + off
    pltpu.async_copy(tab_hbm.at[srcv.at[b]], rows[b], sg[b])

  def gwait(b):
    pltpu.make_async_copy(tab_hbm.at[pl.ds(0, CHUNK)], rows[b], sg[b]).wait()

  def compute(b):
    rw = rows[b]
    for g in range(CHUNK // L):
      s16 = srcv[b, pl.ds(g * L, L)] - off
      d16 = dstv[b, pl.ds(g * L, L)]
      adv = plsc.load_gather(adres, [d16])
      asv = plsc.load_gather(asres, [s16])
      alpha = asv + adv
      alpha = jnp.where(alpha > 0, alpha, 0.2 * alpha)
      ex = jnp.exp(alpha)
      for t in range(L):
        e = g * L + t
        bx = _dg(ex, zero16 + t)
        for k in range(2):
          outv[e, pl.ds(k * L, L)] = rw[e, pl.ds(k * L, L)] * bx
        outv[e, pl.ds(32, L)] = jnp.where(lane < 1, bx, 0.0)
    pltpu.sync_copy(outv, acc.at[dstv.at[b]], add=True)

  prefetch(0, 0)

  def it(i, _):
    prefetch(2 * i + 1, 1)
    gwait(0)
    compute(0)

    @pl.when(i < LOOPN - 1)
    def _():
      prefetch(2 * i + 2, 0)

    gwait(1)
    compute(1)
    return 0

  lax.fori_loop(0, LOOPN, it, 0)
  plsc.subcore_barrier()
  _dump_acc(zbuf, acc, out, cid, sid * RPT)


def kernel(x, edge_index, batch, W1, att_src1, att_dst1, b1,
           W2, att_src2, att_dst2, b2, lin1_W, lin1_b, lin2_W, lin2_b):
  src = edge_index[0]
  dst = edge_index[1]
  eye8 = jnp.eye(8, dtype=F32)
  as1 = (att_src1[0][:, :, None] * eye8[:, None, :]).reshape(64, 8)
  ad1 = (att_dst1[0][:, :, None] * eye8[:, None, :]).reshape(64, 8)
  was2 = att_src2[0, 0].reshape(1, 64)
  wad2 = att_dst2[0, 0].reshape(1, 64)
  b8 = jnp.repeat(eye8, 8, axis=1)

  RB = 2000
  tab1, adst1 = pl.pallas_call(
      _tca_body,
      grid=(NN // RB,),
      in_specs=[pl.BlockSpec((RB, 128), lambda i: (i, 0)),
                pl.BlockSpec((128, 64), lambda i: (0, 0)),
                pl.BlockSpec((64, 8), lambda i: (0, 0)),
                pl.BlockSpec((64, 8), lambda i: (0, 0))],
      out_specs=[pl.BlockSpec((2, RB, TW1), lambda i: (0, i, 0)),
                 pl.BlockSpec((RB, 8), lambda i: (i, 0))],
      out_shape=[jax.ShapeDtypeStruct((2, NN, TW1), F32),
                 jax.ShapeDtypeStruct((NN, 8), F32)],
  )(x, W1, as1, ad1)

  p1 = _sc_layer1(tab1.reshape(2 * NN, TW1), adst1.reshape(NN * 8), src, dst)

  tab2, asrc2, adst2 = pl.pallas_call(
      _tcb_body,
      grid=(NN // RB,),
      in_specs=[pl.BlockSpec((2, RB, AW), lambda i: (0, i, 0)),
                pl.BlockSpec((1, 64), lambda i: (0, 0)),
                pl.BlockSpec((8, 64), lambda i: (0, 0)),
                pl.BlockSpec((64, 64), lambda i: (0, 0)),
                pl.BlockSpec((1, 64), lambda i: (0, 0)),
                pl.BlockSpec((1, 64), lambda i: (0, 0))],
      out_specs=[pl.BlockSpec((2, RB, TW2), lambda i: (0, i, 0)),
                 pl.BlockSpec((RB, 1), lambda i: (i, 0)),
                 pl.BlockSpec((RB, 1), lambda i: (i, 0))],
      out_shape=[jax.ShapeDtypeStruct((2, NN, TW2), F32),
                 jax.ShapeDtypeStruct((NN, 1), F32),
                 jax.ShapeDtypeStruct((NN, 1), F32)],
  )(p1, b1.reshape(1, 64), b8, W2, was2, wad2)

  p2 = _sc_layer2(tab2.reshape(2 * NN, TW2), asrc2.reshape(NN),
                  adst2.reshape(NN), src, dst)

  out = pl.pallas_call(
      _tcc_body,
      grid=(NN // RB,),
      in_specs=[pl.BlockSpec((2, RB, AW), lambda i: (0, i, 0)),
                pl.BlockSpec((RB, 1), lambda i: (i, 0)),
                pl.BlockSpec((1, 64), lambda i: (0, 0)),
                pl.BlockSpec((64, 128), lambda i: (0, 0)),
                pl.BlockSpec((1, 128), lambda i: (0, 0)),
                pl.BlockSpec((128, 1), lambda i: (0, 0)),
                pl.BlockSpec((1, 1), lambda i: (0, 0))],
      out_specs=pl.BlockSpec((NG, 1), lambda i: (0, 0)),
      out_shape=jax.ShapeDtypeStruct((NG, 1), F32),
      scratch_shapes=[pltpu.VMEM((NG, 64), F32), pltpu.VMEM((NG, 1), F32)],
  )(p2, batch.reshape(NN, 1), b2.reshape(1, 64),
    lin1_W, lin1_b.reshape(1, 128), lin2_W, lin2_b.reshape(1, 1))
  return out


# speedup vs baseline: 49.9754x; 1.0343x over previous
import functools

import jax
import jax.numpy as jnp
from jax import lax
from jax.experimental import pallas as pl
from jax.experimental.pallas import tpu as pltpu
from jax.experimental.pallas import tpu_sc as plsc

NN = 10000
EE = 320000
NG = 32
F32 = jnp.float32

NC, NS, L = 2, 16, 16
EPT = EE // NS
CHUNK = 80
NCHUNK = EPT // CHUNK
LOOPN = NCHUNK // 2
TW1 = 40
TW2 = 32
AW = 48
NP = 10240
RPT = NP // NS
ZR = 64


def _tca_body(x_ref, w1_ref, as_ref, ad_ref, tab_ref, adst_ref):
  xw = jnp.dot(x_ref[...], w1_ref[...], preferred_element_type=F32)
  asrc = jnp.dot(xw, as_ref[...], preferred_element_type=F32, precision=lax.Precision.HIGHEST)
  tab_ref[0] = jnp.concatenate([xw[:, 0:32], asrc], axis=1)
  tab_ref[1] = jnp.concatenate([xw[:, 32:64], asrc], axis=1)
  adst_ref[...] = jnp.dot(xw, ad_ref[...], preferred_element_type=F32, precision=lax.Precision.HIGHEST)


def _tcb_body(q_ref, b1r, b8r, w2_ref, was2_ref, wad2_ref,
              tab_ref, asrc2_ref, adst2_ref):
  num = jnp.concatenate([q_ref[0, :, 0:32], q_ref[1, :, 0:32]], axis=1)
  den = q_ref[0, :, 32:40]
  denb = jnp.dot(den, b8r[...], preferred_element_type=F32, precision=lax.Precision.HIGHEST)
  v = num / (denb + 1e-16) + b1r[...]
  h1o = jnp.where(v > 0, v, jnp.exp(v) - 1.0)
  h2 = jnp.dot(h1o, w2_ref[...], preferred_element_type=F32)
  tab_ref[0] = h2[:, 0:32]
  tab_ref[1] = h2[:, 32:64]
  asrc2_ref[...] = jnp.sum(h2 * was2_ref[...], axis=1, keepdims=True)
  adst2_ref[...] = jnp.sum(h2 * wad2_ref[...], axis=1, keepdims=True)


def _tcc_body(q_ref, batch_ref, b2r, l1w, l1b, l2w, l2b, out_ref,
              pool_ref, cnt_ref):
  i = pl.program_id(0)

  @pl.when(i == 0)
  def _():
    pool_ref[...] = jnp.zeros((NG, 64), F32)
    cnt_ref[...] = jnp.zeros((NG, 1), F32)

  num = jnp.concatenate([q_ref[0, :, 0:32], q_ref[1, :, 0:32]], axis=1)
  den = q_ref[0, :, 32:33]
  h2o = num / (den + 1e-16) + b2r[...]
  b = batch_ref[...]
  gid = lax.broadcasted_iota(jnp.int32, (1, NG), 1)
  ot = (b == gid).astype(F32)
  dn = (((0,), (0,)), ((), ()))
  pool_ref[...] += lax.dot_general(ot, h2o, dn, preferred_element_type=F32, precision=lax.Precision.HIGHEST)
  cnt_ref[...] += lax.dot_general(ot, jnp.ones((h2o.shape[0], 1), F32), dn,
                                  preferred_element_type=F32, precision=lax.Precision.HIGHEST)

  @pl.when(i == pl.num_programs(0) - 1)
  def _():
    mean = pool_ref[...] / jnp.maximum(cnt_ref[...], 1.0)
    o1 = jnp.dot(mean, l1w[...], preferred_element_type=F32) + l1b[...]
    o1 = jnp.where(o1 > 0, o1, jnp.exp(o1) - 1.0)
    out_ref[...] = jnp.dot(o1, l2w[...], preferred_element_type=F32) + l2b[...]


def _dg(v, idx):
  return v.at[idx].get(mode="promise_in_bounds")


def _zero_acc(zbuf, acc, base):
  zv = jnp.zeros((L,), F32)
  def zrow(r, _):
    for k in range(AW // L):
      zbuf[r, pl.ds(k * L, L)] = zv
    return 0
  lax.fori_loop(0, ZR, zrow, 0)
  for k in range(RPT // ZR):
    pltpu.sync_copy(zbuf, acc.at[pl.ds(base + k * ZR, ZR), :])


def _dump_acc(zbuf, acc, out_hbm, cid, base):
  for k in range(RPT // ZR):
    pltpu.sync_copy(acc.at[pl.ds(base + k * ZR, ZR), :], zbuf)
    pltpu.sync_copy(zbuf, out_hbm.at[cid, pl.ds(base + k * ZR, ZR), :])


_sc_mesh = plsc.VectorSubcoreMesh(core_axis_name="c", subcore_axis_name="s")
_sc_params = pltpu.CompilerParams(needs_layout_passes=False,
                                  use_tc_tiling_on_sc=False)


@functools.partial(
    pl.kernel,
    out_type=pltpu.HBM((NC, NP, AW), F32),
    mesh=_sc_mesh,
    scratch_types=[
        pltpu.VMEM((2, CHUNK), jnp.int32),
        pltpu.VMEM((2, CHUNK), jnp.int32),
        pltpu.VMEM((CHUNK, TW1), F32),
        pltpu.VMEM((CHUNK, TW1), F32),
        pltpu.VMEM((CHUNK, AW), F32),
        pltpu.VMEM((CHUNK, AW), F32),
        pltpu.VMEM((2, CHUNK), jnp.int32),
        pltpu.VMEM((NN * 8,), F32),
        pltpu.VMEM((ZR, AW), F32),
        pltpu.VMEM_SHARED((NP, AW), F32),
        pltpu.SemaphoreType.DMA,
        pltpu.SemaphoreType.DMA,
        pltpu.SemaphoreType.DMA,
        pltpu.SemaphoreType.DMA,
    ],
    compiler_params=_sc_params,
)
def _sc_layer1(tab_hbm, adst1_hbm, src_hbm, dst_hbm, out,
               srcv, dstv, rows0, rows1, outv0, outv1, dst2, adres, zbuf, acc,
               sg0, sg1, ss0, ss1):
  cid = lax.axis_index("c")
  sid = lax.axis_index("s")
  lane = lax.iota(jnp.int32, L)
  row2 = lane >> 3
  col8 = lane & 7
  rows = (rows0, rows1)
  outv = (outv0, outv1)
  sg = (sg0, sg1)
  ss = (ss0, ss1)
  tbase = sid * EPT
  off = cid * NN

  _zero_acc(zbuf, acc, sid * RPT)
  pltpu.sync_copy(adst1_hbm, adres)
  plsc.subcore_barrier()

  def prefetch(c, b):
    eb = tbase + c * CHUNK
    pltpu.sync_copy(src_hbm.at[pl.ds(eb, CHUNK)], srcv.at[b])
    pltpu.sync_copy(dst_hbm.at[pl.ds(eb, CHUNK)], dstv.at[b])
    for j in range(CHUNK // L):
      srcv[b, pl.ds(j * L, L)] = srcv[b, pl.ds(j * L, L)] + off
    pltpu.async_copy(tab_hbm.at[srcv.at[b]], rows[b], sg[b])

  def gwait(b):
    pltpu.make_async_copy(tab_hbm.at[pl.ds(0, CHUNK)], rows[b], sg[b]).wait()

  def swait(b):
    pltpu.make_async_copy(out.at[cid, pl.ds(0, CHUNK), :], outv[b],
                          ss[b]).wait()

  def compute(b):
    rw, ov = rows[b], outv[b]
    for j in range(CHUNK // L):
      dst2[b, pl.ds(j * L, L)] = dstv[b, pl.ds(j * L, L)]
    for g in range(CHUNK // L):
      d16 = dstv[b, pl.ds(g * L, L)]
      for p in range(L // 2):
        e0 = g * L + 2 * p
        rid = _dg(d16, row2 + 2 * p)
        adv = plsc.load_gather(adres, [rid * 8 + col8])
        va = rw[e0, pl.ds(24, L)]
        vb = rw[e0 + 1, pl.ds(24, L)]
        asv = jnp.where(lane < 8, _dg(va, col8 + 8), vb)
        alpha = asv + adv
        alpha = jnp.where(alpha > 0, alpha, 0.2 * alpha)
        ex = jnp.exp(alpha)
        for k in range(2):
          bA = _dg(ex, row2 + (2 * k + 4 * cid))
          ov[e0, pl.ds(k * L, L)] = rw[e0, pl.ds(k * L, L)] * bA
          bB = _dg(ex, row2 + (2 * k + 4 * cid + 8))
          ov[e0 + 1, pl.ds(k * L, L)] = rw[e0 + 1, pl.ds(k * L, L)] * bB
        ov[e0, pl.ds(32, L)] = jnp.where(lane < 8, ex, 0.0)
        exb = _dg(ex, col8 + 8)
        ov[e0 + 1, pl.ds(32, L)] = jnp.where(lane < 8, exb, 0.0)
    pltpu.async_copy(ov, acc.at[dst2.at[b]], ss[b], add=True)

  prefetch(0, 0)

  def it(i, _):
    prefetch(2 * i + 1, 1)
    gwait(0)

    @pl.when(i > 0)
    def _():
      swait(0)

    compute(0)

    @pl.when(i < LOOPN - 1)
    def _():
      prefetch(2 * i + 2, 0)

    gwait(1)

    @pl.when(i > 0)
    def _():
      swait(1)

    compute(1)
    return 0

  lax.fori_loop(0, LOOPN, it, 0)
  swait(0)
  swait(1)
  plsc.subcore_barrier()
  _dump_acc(zbuf, acc, out, cid, sid * RPT)


@functools.partial(
    pl.kernel,
    out_type=pltpu.HBM((NC, NP, AW), F32),
    mesh=_sc_mesh,
    scratch_types=[
        pltpu.VMEM((2, CHUNK), jnp.int32),
        pltpu.VMEM((2, CHUNK), jnp.int32),
        pltpu.VMEM((CHUNK, TW2), F32),
        pltpu.VMEM((CHUNK, TW2), F32),
        pltpu.VMEM((CHUNK, AW), F32),
        pltpu.VMEM((CHUNK, AW), F32),
        pltpu.VMEM((2, CHUNK), jnp.int32),
        pltpu.VMEM((NN,), F32),
        pltpu.VMEM((NN,), F32),
        pltpu.VMEM((ZR, AW), F32),
        pltpu.VMEM_SHARED((NP, AW), F32),
        pltpu.SemaphoreType.DMA,
        pltpu.SemaphoreType.DMA,
        pltpu.SemaphoreType.DMA,
        pltpu.SemaphoreType.DMA,
    ],
    compiler_params=_sc_params,
)
def _sc_layer2(tab_hbm, asrc2_hbm, adst2_hbm, src_hbm, dst_hbm, out,
               srcv, dstv, rows0, rows1, outv0, outv1, dst2, asres, adres,
               zbuf, acc, sg0, sg1, ss0, ss1):
  cid = lax.axis_index("c")
  sid = lax.axis_index("s")
  lane = lax.iota(jnp.int32, L)
  zero16 = lane * 0
  rows = (rows0, rows1)
  outv = (outv0, outv1)
  sg = (sg0, sg1)
  ss = (ss0, ss1)
  tbase = sid * EPT
  off = cid * NN

  _zero_acc(zbuf, acc, sid * RPT)
  pltpu.sync_copy(asrc2_hbm, asres)
  pltpu.sync_copy(adst2_hbm, adres)
  plsc.subcore_barrier()

  def prefetch(c, b):
    eb = tbase + c * CHUNK
    pltpu.sync_copy(src_hbm.at[pl.ds(eb, CHUNK)], srcv.at[b])
    pltpu.sync_copy(dst_hbm.at[pl.ds(eb, CHUNK)], dstv.at[b])
    for j in range(CHUNK // L):
      srcv[b, pl.ds(j * L, L)] = srcv[b, pl.ds(j * L, L)] + off
    pltpu.async_copy(tab_hbm.at[srcv.at[b]], rows[b], sg[b])

  def gwait(b):
    pltpu.make_async_copy(tab_hbm.at[pl.ds(0, CHUNK)], rows[b], sg[b]).wait()

  def swait(b):
    pltpu.make_async_copy(out.at[cid, pl.ds(0, CHUNK), :], outv[b],
                          ss[b]).wait()

  def compute(b):
    rw, ov = rows[b], outv[b]
    for j in range(CHUNK // L):
      dst2[b, pl.ds(j * L, L)] = dstv[b, pl.ds(j * L, L)]
    for g in range(CHUNK // L):
      s16 = srcv[b, pl.ds(g * L, L)] - off
      d16 = dstv[b, pl.ds(g * L, L)]
      adv = plsc.load_gather(adres, [d16])
      asv = plsc.load_gather(asres, [s16])
      alpha = asv + adv
      alpha = jnp.where(alpha > 0, alpha, 0.2 * alpha)
      ex = jnp.exp(alpha)
      for t in range(L):
        e = g * L + t
        bx = _dg(ex, zero16 + t)
        for k in range(2):
          ov[e, pl.ds(k * L, L)] = rw[e, pl.ds(k * L, L)] * bx
        ov[e, pl.ds(32, L)] = jnp.where(lane < 1, bx, 0.0)
    pltpu.async_copy(ov, acc.at[dst2.at[b]], ss[b], add=True)

  prefetch(0, 0)

  def it(i, _):
    prefetch(2 * i + 1, 1)
    gwait(0)

    @pl.when(i > 0)
    def _():
      swait(0)

    compute(0)

    @pl.when(i < LOOPN - 1)
    def _():
      prefetch(2 * i + 2, 0)

    gwait(1)

    @pl.when(i > 0)
    def _():
      swait(1)

    compute(1)
    return 0

  lax.fori_loop(0, LOOPN, it, 0)
  swait(0)
  swait(1)
  plsc.subcore_barrier()
  _dump_acc(zbuf, acc, out, cid, sid * RPT)


def kernel(x, edge_index, batch, W1, att_src1, att_dst1, b1,
           W2, att_src2, att_dst2, b2, lin1_W, lin1_b, lin2_W, lin2_b):
  src = edge_index[0]
  dst = edge_index[1]
  eye8 = jnp.eye(8, dtype=F32)
  as1 = (att_src1[0][:, :, None] * eye8[:, None, :]).reshape(64, 8)
  ad1 = (att_dst1[0][:, :, None] * eye8[:, None, :]).reshape(64, 8)
  was2 = att_src2[0, 0].reshape(1, 64)
  wad2 = att_dst2[0, 0].reshape(1, 64)
  b8 = jnp.repeat(eye8, 8, axis=1)

  RB = 2000
  tab1, adst1 = pl.pallas_call(
      _tca_body,
      grid=(NN // RB,),
      in_specs=[pl.BlockSpec((RB, 128), lambda i: (i, 0)),
                pl.BlockSpec((128, 64), lambda i: (0, 0)),
                pl.BlockSpec((64, 8), lambda i: (0, 0)),
                pl.BlockSpec((64, 8), lambda i: (0, 0))],
      out_specs=[pl.BlockSpec((2, RB, TW1), lambda i: (0, i, 0)),
                 pl.BlockSpec((RB, 8), lambda i: (i, 0))],
      out_shape=[jax.ShapeDtypeStruct((2, NN, TW1), F32),
                 jax.ShapeDtypeStruct((NN, 8), F32)],
  )(x, W1, as1, ad1)

  p1 = _sc_layer1(tab1.reshape(2 * NN, TW1), adst1.reshape(NN * 8), src, dst)

  tab2, asrc2, adst2 = pl.pallas_call(
      _tcb_body,
      grid=(NN // RB,),
      in_specs=[pl.BlockSpec((2, RB, AW), lambda i: (0, i, 0)),
                pl.BlockSpec((1, 64), lambda i: (0, 0)),
                pl.BlockSpec((8, 64), lambda i: (0, 0)),
                pl.BlockSpec((64, 64), lambda i: (0, 0)),
                pl.BlockSpec((1, 64), lambda i: (0, 0)),
                pl.BlockSpec((1, 64), lambda i: (0, 0))],
      out_specs=[pl.BlockSpec((2, RB, TW2), lambda i: (0, i, 0)),
                 pl.BlockSpec((RB, 1), lambda i: (i, 0)),
                 pl.BlockSpec((RB, 1), lambda i: (i, 0))],
      out_shape=[jax.ShapeDtypeStruct((2, NN, TW2), F32),
                 jax.ShapeDtypeStruct((NN, 1), F32),
                 jax.ShapeDtypeStruct((NN, 1), F32)],
  )(p1, b1.reshape(1, 64), b8, W2, was2, wad2)

  p2 = _sc_layer2(tab2.reshape(2 * NN, TW2), asrc2.reshape(NN),
                  adst2.reshape(NN), src, dst)

  out = pl.pallas_call(
      _tcc_body,
      grid=(NN // RB,),
      in_specs=[pl.BlockSpec((2, RB, AW), lambda i: (0, i, 0)),
                pl.BlockSpec((RB, 1), lambda i: (i, 0)),
                pl.BlockSpec((1, 64), lambda i: (0, 0)),
                pl.BlockSpec((64, 128), lambda i: (0, 0)),
                pl.BlockSpec((1, 128), lambda i: (0, 0)),
                pl.BlockSpec((128, 1), lambda i: (0, 0)),
                pl.BlockSpec((1, 1), lambda i: (0, 0))],
      out_specs=pl.BlockSpec((NG, 1), lambda i: (0, 0)),
      out_shape=jax.ShapeDtypeStruct((NG, 1), F32),
      scratch_shapes=[pltpu.VMEM((NG, 64), F32), pltpu.VMEM((NG, 1), F32)],
  )(p2, batch.reshape(NN, 1), b2.reshape(1, 64),
    lin1_W, lin1_b.reshape(1, 128), lin2_W, lin2_b.reshape(1, 1))
  return out


# speedup vs baseline: 50.8540x; 1.0176x over previous
import functools

import jax
import jax.numpy as jnp
from jax import lax
from jax.experimental import pallas as pl
from jax.experimental.pallas import tpu as pltpu
from jax.experimental.pallas import tpu_sc as plsc

NN = 10000
EE = 320000
NG = 32
F32 = jnp.float32

NC, NS, L = 2, 16, 16
EPT = EE // NS
CHUNK = 80
NCHUNK = EPT // CHUNK
LOOPN = NCHUNK // 2
TW1 = 40
TW2 = 32
AW = 48
NP = 10240
RPT = NP // NS
ZR = 64


def _tca_body(x_ref, w1_ref, as_ref, ad_ref, tab_ref, adst_ref):
  xw = jnp.dot(x_ref[...], w1_ref[...], preferred_element_type=F32)
  asrc = jnp.dot(xw, as_ref[...], preferred_element_type=F32, precision=lax.Precision.HIGHEST)
  tab_ref[0] = jnp.concatenate([xw[:, 0:32], asrc], axis=1)
  tab_ref[1] = jnp.concatenate([xw[:, 32:64], asrc], axis=1)
  adst_ref[...] = jnp.dot(xw, ad_ref[...], preferred_element_type=F32, precision=lax.Precision.HIGHEST)


def _tcb_body(q_ref, b1r, b8r, w2_ref, was2_ref, wad2_ref,
              tab_ref, asrc2_ref, adst2_ref):
  num = jnp.concatenate([q_ref[0, :, 0:32], q_ref[1, :, 0:32]], axis=1)
  den = q_ref[0, :, 32:40]
  denb = jnp.dot(den, b8r[...], preferred_element_type=F32, precision=lax.Precision.HIGHEST)
  v = num / (denb + 1e-16) + b1r[...]
  h1o = jnp.where(v > 0, v, jnp.exp(v) - 1.0)
  h2 = jnp.dot(h1o, w2_ref[...], preferred_element_type=F32)
  tab_ref[0] = h2[:, 0:32]
  tab_ref[1] = h2[:, 32:64]
  asrc2_ref[...] = jnp.sum(h2 * was2_ref[...], axis=1, keepdims=True)
  adst2_ref[...] = jnp.sum(h2 * wad2_ref[...], axis=1, keepdims=True)


def _tcc_body(q_ref, batch_ref, b2r, l1w, l1b, l2w, l2b, out_ref,
              pool_ref, cnt_ref):
  i = pl.program_id(0)

  @pl.when(i == 0)
  def _():
    pool_ref[...] = jnp.zeros((NG, 64), F32)
    cnt_ref[...] = jnp.zeros((NG, 1), F32)

  num = jnp.concatenate([q_ref[0, :, 0:32], q_ref[1, :, 0:32]], axis=1)
  den = q_ref[0, :, 32:33]
  h2o = num / (den + 1e-16) + b2r[...]
  b = batch_ref[...]
  gid = lax.broadcasted_iota(jnp.int32, (1, NG), 1)
  ot = (b == gid).astype(F32)
  dn = (((0,), (0,)), ((), ()))
  pool_ref[...] += lax.dot_general(ot, h2o, dn, preferred_element_type=F32, precision=lax.Precision.HIGHEST)
  cnt_ref[...] += lax.dot_general(ot, jnp.ones((h2o.shape[0], 1), F32), dn,
                                  preferred_element_type=F32, precision=lax.Precision.HIGHEST)

  @pl.when(i == pl.num_programs(0) - 1)
  def _():
    mean = pool_ref[...] / jnp.maximum(cnt_ref[...], 1.0)
    o1 = jnp.dot(mean, l1w[...], preferred_element_type=F32) + l1b[...]
    o1 = jnp.where(o1 > 0, o1, jnp.exp(o1) - 1.0)
    out_ref[...] = jnp.dot(o1, l2w[...], preferred_element_type=F32) + l2b[...]


def _dg(v, idx):
  return v.at[idx].get(mode="promise_in_bounds")


def _zero_acc(zbuf, acc, base):
  zv = jnp.zeros((L,), F32)
  def zrow(r, _):
    for k in range(AW // L):
      zbuf[r, pl.ds(k * L, L)] = zv
    return 0
  lax.fori_loop(0, ZR, zrow, 0)
  for k in range(RPT // ZR):
    pltpu.sync_copy(zbuf, acc.at[pl.ds(base + k * ZR, ZR), :])


def _dump_acc(zbuf, acc, out_hbm, cid, base):
  for k in range(RPT // ZR):
    pltpu.sync_copy(acc.at[pl.ds(base + k * ZR, ZR), :], zbuf)
    pltpu.sync_copy(zbuf, out_hbm.at[cid, pl.ds(base + k * ZR, ZR), :])


_sc_mesh = plsc.VectorSubcoreMesh(core_axis_name="c", subcore_axis_name="s")
_sc_params = pltpu.CompilerParams(needs_layout_passes=False,
                                  use_tc_tiling_on_sc=False)


@functools.partial(
    pl.kernel,
    out_type=pltpu.HBM((NC, NP, AW), F32),
    mesh=_sc_mesh,
    scratch_types=[
        pltpu.VMEM((2, CHUNK), jnp.int32),
        pltpu.VMEM((2, CHUNK), jnp.int32),
        pltpu.VMEM((CHUNK, TW1), F32),
        pltpu.VMEM((CHUNK, TW1), F32),
        pltpu.VMEM((CHUNK, AW), F32),
        pltpu.VMEM((CHUNK, AW), F32),
        pltpu.VMEM((2, CHUNK), jnp.int32),
        pltpu.VMEM((CHUNK * 8,), F32),
        pltpu.VMEM((NN * 8,), F32),
        pltpu.VMEM((ZR, AW), F32),
        pltpu.VMEM_SHARED((NP, AW), F32),
        pltpu.SemaphoreType.DMA,
        pltpu.SemaphoreType.DMA,
        pltpu.SemaphoreType.DMA,
        pltpu.SemaphoreType.DMA,
    ],
    compiler_params=_sc_params,
)
def _sc_layer1(tab_hbm, adst1_hbm, src_hbm, dst_hbm, out,
               srcv, dstv, rows0, rows1, outv0, outv1, dst2, exbuf, adres,
               zbuf, acc, sg0, sg1, ss0, ss1):
  cid = lax.axis_index("c")
  sid = lax.axis_index("s")
  lane = lax.iota(jnp.int32, L)
  row2 = lane >> 3
  col8 = lane & 7
  rows = (rows0, rows1)
  outv = (outv0, outv1)
  sg = (sg0, sg1)
  ss = (ss0, ss1)
  tbase = sid * EPT
  off = cid * NN

  _zero_acc(zbuf, acc, sid * RPT)
  pltpu.sync_copy(adst1_hbm, adres)
  plsc.subcore_barrier()

  def prefetch(c, b):
    eb = tbase + c * CHUNK
    pltpu.sync_copy(src_hbm.at[pl.ds(eb, CHUNK)], srcv.at[b])
    pltpu.sync_copy(dst_hbm.at[pl.ds(eb, CHUNK)], dstv.at[b])
    for j in range(CHUNK // L):
      srcv[b, pl.ds(j * L, L)] = srcv[b, pl.ds(j * L, L)] + off
    pltpu.async_copy(tab_hbm.at[srcv.at[b]], rows[b], sg[b])

  def gwait(b):
    pltpu.make_async_copy(tab_hbm.at[pl.ds(0, CHUNK)], rows[b], sg[b]).wait()

  def swait(b):
    pltpu.make_async_copy(out.at[cid, pl.ds(0, CHUNK), :], outv[b],
                          ss[b]).wait()

  def compute(b):
    rw, ov = rows[b], outv[b]
    for j in range(CHUNK // L):
      dst2[b, pl.ds(j * L, L)] = dstv[b, pl.ds(j * L, L)]
    for g in range(CHUNK // L):
      d16 = dstv[b, pl.ds(g * L, L)]
      for p in range(L // 2):
        e0 = g * L + 2 * p
        rid = _dg(d16, row2 + 2 * p)
        adv = plsc.load_gather(adres, [rid * 8 + col8])
        va = rw[e0, pl.ds(24, L)]
        vb = rw[e0 + 1, pl.ds(24, L)]
        asv = jnp.where(lane < 8, _dg(va, col8 + 8), vb)
        alpha = asv + adv
        alpha = jnp.where(alpha > 0, alpha, 0.2 * alpha)
        exbuf[pl.ds(e0 * 8, L)] = jnp.exp(alpha)
    for g in range(CHUNK // L):
      for p in range(L // 2):
        e0 = g * L + 2 * p
        ex = exbuf[pl.ds(e0 * 8, L)]
        for k in range(2):
          bA = _dg(ex, row2 + (2 * k + 4 * cid))
          ov[e0, pl.ds(k * L, L)] = rw[e0, pl.ds(k * L, L)] * bA
          bB = _dg(ex, row2 + (2 * k + 4 * cid + 8))
          ov[e0 + 1, pl.ds(k * L, L)] = rw[e0 + 1, pl.ds(k * L, L)] * bB
        ov[e0, pl.ds(32, L)] = jnp.where(lane < 8, ex, 0.0)
        exb = _dg(ex, col8 + 8)
        ov[e0 + 1, pl.ds(32, L)] = jnp.where(lane < 8, exb, 0.0)
    pltpu.async_copy(ov, acc.at[dst2.at[b]], ss[b], add=True)

  prefetch(0, 0)

  def it(i, _):
    prefetch(2 * i + 1, 1)
    gwait(0)

    @pl.when(i > 0)
    def _():
      swait(0)

    compute(0)

    @pl.when(i < LOOPN - 1)
    def _():
      prefetch(2 * i + 2, 0)

    gwait(1)

    @pl.when(i > 0)
    def _():
      swait(1)

    compute(1)
    return 0

  lax.fori_loop(0, LOOPN, it, 0)
  swait(0)
  swait(1)
  plsc.subcore_barrier()
  _dump_acc(zbuf, acc, out, cid, sid * RPT)


@functools.partial(
    pl.kernel,
    out_type=pltpu.HBM((NC, NP, AW), F32),
    mesh=_sc_mesh,
    scratch_types=[
        pltpu.VMEM((2, CHUNK), jnp.int32),
        pltpu.VMEM((2, CHUNK), jnp.int32),
        pltpu.VMEM((CHUNK, TW2), F32),
        pltpu.VMEM((CHUNK, TW2), F32),
        pltpu.VMEM((CHUNK, AW), F32),
        pltpu.VMEM((CHUNK, AW), F32),
        pltpu.VMEM((2, CHUNK), jnp.int32),
        pltpu.VMEM((NN,), F32),
        pltpu.VMEM((NN,), F32),
        pltpu.VMEM((ZR, AW), F32),
        pltpu.VMEM_SHARED((NP, AW), F32),
        pltpu.SemaphoreType.DMA,
        pltpu.SemaphoreType.DMA,
        pltpu.SemaphoreType.DMA,
        pltpu.SemaphoreType.DMA,
    ],
    compiler_params=_sc_params,
)
def _sc_layer2(tab_hbm, asrc2_hbm, adst2_hbm, src_hbm, dst_hbm, out,
               srcv, dstv, rows0, rows1, outv0, outv1, dst2, asres, adres,
               zbuf, acc, sg0, sg1, ss0, ss1):
  cid = lax.axis_index("c")
  sid = lax.axis_index("s")
  lane = lax.iota(jnp.int32, L)
  zero16 = lane * 0
  rows = (rows0, rows1)
  outv = (outv0, outv1)
  sg = (sg0, sg1)
  ss = (ss0, ss1)
  tbase = sid * EPT
  off = cid * NN

  _zero_acc(zbuf, acc, sid * RPT)
  pltpu.sync_copy(asrc2_hbm, asres)
  pltpu.sync_copy(adst2_hbm, adres)
  plsc.subcore_barrier()

  def prefetch(c, b):
    eb = tbase + c * CHUNK
    pltpu.sync_copy(src_hbm.at[pl.ds(eb, CHUNK)], srcv.at[b])
    pltpu.sync_copy(dst_hbm.at[pl.ds(eb, CHUNK)], dstv.at[b])
    for j in range(CHUNK // L):
      srcv[b, pl.ds(j * L, L)] = srcv[b, pl.ds(j * L, L)] + off
    pltpu.async_copy(tab_hbm.at[srcv.at[b]], rows[b], sg[b])

  def gwait(b):
    pltpu.make_async_copy(tab_hbm.at[pl.ds(0, CHUNK)], rows[b], sg[b]).wait()

  def swait(b):
    pltpu.make_async_copy(out.at[cid, pl.ds(0, CHUNK), :], outv[b],
                          ss[b]).wait()

  def compute(b):
    rw, ov = rows[b], outv[b]
    for j in range(CHUNK // L):
      dst2[b, pl.ds(j * L, L)] = dstv[b, pl.ds(j * L, L)]
    for g in range(CHUNK // L):
      s16 = srcv[b, pl.ds(g * L, L)] - off
      d16 = dstv[b, pl.ds(g * L, L)]
      adv = plsc.load_gather(adres, [d16])
      asv = plsc.load_gather(asres, [s16])
      alpha = asv + adv
      alpha = jnp.where(alpha > 0, alpha, 0.2 * alpha)
      ex = jnp.exp(alpha)
      for t in range(L):
        e = g * L + t
        bx = _dg(ex, zero16 + t)
        for k in range(2):
          ov[e, pl.ds(k * L, L)] = rw[e, pl.ds(k * L, L)] * bx
        ov[e, pl.ds(32, L)] = jnp.where(lane < 1, bx, 0.0)
    pltpu.async_copy(ov, acc.at[dst2.at[b]], ss[b], add=True)

  prefetch(0, 0)

  def it(i, _):
    prefetch(2 * i + 1, 1)
    gwait(0)

    @pl.when(i > 0)
    def _():
      swait(0)

    compute(0)

    @pl.when(i < LOOPN - 1)
    def _():
      prefetch(2 * i + 2, 0)

    gwait(1)

    @pl.when(i > 0)
    def _():
      swait(1)

    compute(1)
    return 0

  lax.fori_loop(0, LOOPN, it, 0)
  swait(0)
  swait(1)
  plsc.subcore_barrier()
  _dump_acc(zbuf, acc, out, cid, sid * RPT)


def kernel(x, edge_index, batch, W1, att_src1, att_dst1, b1,
           W2, att_src2, att_dst2, b2, lin1_W, lin1_b, lin2_W, lin2_b):
  src = edge_index[0]
  dst = edge_index[1]
  eye8 = jnp.eye(8, dtype=F32)
  as1 = (att_src1[0][:, :, None] * eye8[:, None, :]).reshape(64, 8)
  ad1 = (att_dst1[0][:, :, None] * eye8[:, None, :]).reshape(64, 8)
  was2 = att_src2[0, 0].reshape(1, 64)
  wad2 = att_dst2[0, 0].reshape(1, 64)
  b8 = jnp.repeat(eye8, 8, axis=1)

  RB = 2000
  tab1, adst1 = pl.pallas_call(
      _tca_body,
      grid=(NN // RB,),
      in_specs=[pl.BlockSpec((RB, 128), lambda i: (i, 0)),
                pl.BlockSpec((128, 64), lambda i: (0, 0)),
                pl.BlockSpec((64, 8), lambda i: (0, 0)),
                pl.BlockSpec((64, 8), lambda i: (0, 0))],
      out_specs=[pl.BlockSpec((2, RB, TW1), lambda i: (0, i, 0)),
                 pl.BlockSpec((RB, 8), lambda i: (i, 0))],
      out_shape=[jax.ShapeDtypeStruct((2, NN, TW1), F32),
                 jax.ShapeDtypeStruct((NN, 8), F32)],
  )(x, W1, as1, ad1)

  p1 = _sc_layer1(tab1.reshape(2 * NN, TW1), adst1.reshape(NN * 8), src, dst)

  tab2, asrc2, adst2 = pl.pallas_call(
      _tcb_body,
      grid=(NN // RB,),
      in_specs=[pl.BlockSpec((2, RB, AW), lambda i: (0, i, 0)),
                pl.BlockSpec((1, 64), lambda i: (0, 0)),
                pl.BlockSpec((8, 64), lambda i: (0, 0)),
                pl.BlockSpec((64, 64), lambda i: (0, 0)),
                pl.BlockSpec((1, 64), lambda i: (0, 0)),
                pl.BlockSpec((1, 64), lambda i: (0, 0))],
      out_specs=[pl.BlockSpec((2, RB, TW2), lambda i: (0, i, 0)),
                 pl.BlockSpec((RB, 1), lambda i: (i, 0)),
                 pl.BlockSpec((RB, 1), lambda i: (i, 0))],
      out_shape=[jax.ShapeDtypeStruct((2, NN, TW2), F32),
                 jax.ShapeDtypeStruct((NN, 1), F32),
                 jax.ShapeDtypeStruct((NN, 1), F32)],
  )(p1, b1.reshape(1, 64), b8, W2, was2, wad2)

  p2 = _sc_layer2(tab2.reshape(2 * NN, TW2), asrc2.reshape(NN),
                  adst2.reshape(NN), src, dst)

  out = pl.pallas_call(
      _tcc_body,
      grid=(NN // RB,),
      in_specs=[pl.BlockSpec((2, RB, AW), lambda i: (0, i, 0)),
                pl.BlockSpec((RB, 1), lambda i: (i, 0)),
                pl.BlockSpec((1, 64), lambda i: (0, 0)),
                pl.BlockSpec((64, 128), lambda i: (0, 0)),
                pl.BlockSpec((1, 128), lambda i: (0, 0)),
                pl.BlockSpec((128, 1), lambda i: (0, 0)),
                pl.BlockSpec((1, 1), lambda i: (0, 0))],
      out_specs=pl.BlockSpec((NG, 1), lambda i: (0, 0)),
      out_shape=jax.ShapeDtypeStruct((NG, 1), F32),
      scratch_shapes=[pltpu.VMEM((NG, 64), F32), pltpu.VMEM((NG, 1), F32)],
  )(p2, batch.reshape(NN, 1), b2.reshape(1, 64),
    lin1_W, lin1_b.reshape(1, 128), lin2_W, lin2_b.reshape(1, 1))
  return out
